# baseline (speedup 1.0000x reference)
import jax
import jax.numpy as jnp
from jax.experimental import pallas as pl
from jax.experimental.pallas import tpu as pltpu

_CELLS = 2
_IPS = 4


def _encoder_body(x_ref, bn_scale_ref, bn_shift_ref, w0_ref,
                  w1_ref, o_ref, *pats):
    H = x_ref.shape[1]
    W = x_ref.shape[2]
    C = x_ref.shape[3]
    HW = H * W

    lane = jax.lax.broadcasted_iota(jnp.int32, (H, W, C), 2)
    ones_lane = jnp.where(lane == 0, 1.0, 0.0).astype(jnp.bfloat16)
    for pat in pats:
        pat[...] = jnp.zeros(pat.shape, pat.dtype)
        pat[:, :, 9 * C:10 * C] = ones_lane

    def bn_conv(pat, x2d, bn_row, w):
        scale = bn_scale_ref[bn_row:bn_row + 1, :]
        shift = bn_shift_ref[bn_row:bn_row + 1, :]
        bnx = (x2d * scale + shift).astype(jnp.bfloat16).reshape(H, W, C)
        for kh in range(3):
            r0, r1 = (1, H) if kh == 0 else ((0, H) if kh == 1 else (0, H - 1))
            s0 = r0 + kh - 1
            for kw in range(3):
                t = 3 * kh + kw
                if kw == 0:
                    pat[r0:r1, 1:W, t * C:(t + 1) * C] = \
                        bnx[s0:s0 + (r1 - r0), 0:W - 1, :]
                elif kw == 1:
                    pat[r0:r1, :, t * C:(t + 1) * C] = \
                        bnx[s0:s0 + (r1 - r0), :, :]
                else:
                    pat[r0:r1, 0:W - 1, t * C:(t + 1) * C] = \
                        bnx[s0:s0 + (r1 - r0), 1:W, :]
        return jnp.dot(pat[...].reshape(HW, 10 * C), w,
                       preferred_element_type=jnp.float32)

    cells = [x_ref[i].reshape(HW, C).astype(jnp.float32) for i in range(_IPS)]
    for c in range(_CELLS):
        y0 = [bn_conv(pats[i], cells[i], 2 * c + 0, w0_ref[c])
              for i in range(_IPS)]
        n1 = [jnp.maximum(y0[i][:, :C], 0.0) for i in range(_IPS)]
        y1 = [bn_conv(pats[i], n1[i], 2 * c + 1, w1_ref[c])
              for i in range(_IPS)]
        cells = [y0[i][:, C:] + jnp.maximum(y1[i], 0.0) for i in range(_IPS)]
    for i in range(_IPS):
        o_ref[i] = jnp.sum(cells[i], axis=0, keepdims=True) * (1.0 / HW)


def _head_body(p_ref, hw_ref, hb_ref, o_ref):
    o_ref[...] = jnp.dot(p_ref[...], hw_ref[...],
                         preferred_element_type=jnp.float32) + hb_ref[...]


def kernel(x, bn_scale, bn_shift, w0, b0, w1, b1, head_w, head_b):
    x = jnp.transpose(x, (0, 2, 3, 1)).astype(jnp.bfloat16)
    B, H, W, C = x.shape
    K = head_w.shape[1]
    ten_c = 10 * C

    def pack_w(w, b):
        ext = jnp.zeros((_CELLS, C, w.shape[2]), jnp.float32)
        ext = ext.at[:, 0:1, :].set(b)
        return jnp.concatenate([w, ext], axis=1).astype(jnp.bfloat16)

    pooled = pl.pallas_call(
        _encoder_body,
        out_shape=jax.ShapeDtypeStruct((B, 1, C), jnp.float32),
        grid=(B // _IPS,),
        in_specs=[
            pl.BlockSpec((_IPS, H, W, C), lambda b: (b, 0, 0, 0)),
            pl.BlockSpec((2 * _CELLS, C), lambda b: (0, 0)),
            pl.BlockSpec((2 * _CELLS, C), lambda b: (0, 0)),
            pl.BlockSpec((_CELLS, ten_c, 2 * C), lambda b: (0, 0, 0)),
            pl.BlockSpec((_CELLS, ten_c, C), lambda b: (0, 0, 0)),
        ],
        out_specs=pl.BlockSpec((_IPS, 1, C), lambda b: (b, 0, 0)),
        scratch_shapes=[pltpu.VMEM((H, W, ten_c), jnp.bfloat16)
                        for _ in range(_IPS)],
        compiler_params=pltpu.CompilerParams(dimension_semantics=("parallel",)),
    )(x, bn_scale, bn_shift, pack_w(w0, b0), pack_w(w1, b1))

    logits = pl.pallas_call(
        _head_body,
        out_shape=jax.ShapeDtypeStruct((B, K), jnp.float32),
    )(pooled.reshape(B, C), head_w, head_b)
    return logits


# speedup vs baseline: 1.0658x; 1.0658x over previous
import jax
import jax.numpy as jnp
from jax.experimental import pallas as pl
from jax.experimental.pallas import tpu as pltpu

_CELLS = 2
_IPS = 4


def _encoder_body(x_ref, bn_scale_ref, bn_shift_ref, w0_ref,
                  w1_ref, o_ref, *scratch):
    H = x_ref.shape[1]
    W = x_ref.shape[2]
    C = x_ref.shape[3]
    HW = H * W
    stgs = scratch[:_IPS]
    pats = scratch[_IPS:]

    lane = jax.lax.broadcasted_iota(jnp.int32, (H, W, C), 2)
    ones_lane = jnp.where(lane == 0, 1.0, 0.0).astype(jnp.bfloat16)
    for stg, pat in zip(stgs, pats):
        stg[...] = jnp.zeros(stg.shape, stg.dtype)
        pat[:, :, 9 * C:10 * C] = ones_lane

    def bn_conv(stg, pat, x2d, bn_row, w):
        scale = bn_scale_ref[bn_row:bn_row + 1, :]
        shift = bn_shift_ref[bn_row:bn_row + 1, :]
        bnx = (x2d * scale + shift).astype(jnp.bfloat16).reshape(H, W, C)
        stg[1:H + 1, :, C:2 * C] = bnx
        stg[1:H + 1, 1:W, 0:C] = bnx[:, :W - 1, :]
        stg[1:H + 1, 0:W - 1, 2 * C:3 * C] = bnx[:, 1:, :]
        pat[:, :, 0:3 * C] = stg[0:H, :, :]
        pat[:, :, 3 * C:6 * C] = stg[1:H + 1, :, :]
        pat[:, :, 6 * C:9 * C] = stg[2:H + 2, :, :]
        return jnp.dot(pat[...].reshape(HW, 10 * C), w,
                       preferred_element_type=jnp.float32)

    cells = [x_ref[i].reshape(HW, C) for i in range(_IPS)]
    for c in range(_CELLS):
        y0 = [bn_conv(stgs[i], pats[i], cells[i], 2 * c + 0, w0_ref[c])
              for i in range(_IPS)]
        n1 = [jnp.maximum(y0[i][:, :C], 0.0).astype(jnp.bfloat16)
              for i in range(_IPS)]
        y1 = [bn_conv(stgs[i], pats[i], n1[i], 2 * c + 1, w1_ref[c])
              for i in range(_IPS)]
        cells = [(y0[i][:, C:] + jnp.maximum(y1[i], 0.0)).astype(jnp.bfloat16)
                 for i in range(_IPS)]
    for i in range(_IPS):
        o_ref[i] = (jnp.sum(cells[i].astype(jnp.float32), axis=0,
                            keepdims=True) * (1.0 / HW))


def _head_body(p_ref, hw_ref, hb_ref, o_ref):
    o_ref[...] = jnp.dot(p_ref[...], hw_ref[...],
                         preferred_element_type=jnp.float32) + hb_ref[...]


def kernel(x, bn_scale, bn_shift, w0, b0, w1, b1, head_w, head_b):
    x = jnp.transpose(x, (0, 2, 3, 1)).astype(jnp.bfloat16)
    B, H, W, C = x.shape
    K = head_w.shape[1]
    ten_c = 10 * C

    def pack_w(w, b):
        ext = jnp.zeros((_CELLS, C, w.shape[2]), jnp.float32)
        ext = ext.at[:, 0:1, :].set(b)
        return jnp.concatenate([w, ext], axis=1).astype(jnp.bfloat16)

    pooled = pl.pallas_call(
        _encoder_body,
        out_shape=jax.ShapeDtypeStruct((B, 1, C), jnp.float32),
        grid=(B // _IPS,),
        in_specs=[
            pl.BlockSpec((_IPS, H, W, C), lambda b: (b, 0, 0, 0)),
            pl.BlockSpec((2 * _CELLS, C), lambda b: (0, 0)),
            pl.BlockSpec((2 * _CELLS, C), lambda b: (0, 0)),
            pl.BlockSpec((_CELLS, ten_c, 2 * C), lambda b: (0, 0, 0)),
            pl.BlockSpec((_CELLS, ten_c, C), lambda b: (0, 0, 0)),
        ],
        out_specs=pl.BlockSpec((_IPS, 1, C), lambda b: (b, 0, 0)),
        scratch_shapes=(
            [pltpu.VMEM((H + 2, W, 3 * C), jnp.bfloat16) for _ in range(_IPS)]
            + [pltpu.VMEM((H, W, ten_c), jnp.bfloat16) for _ in range(_IPS)]),
        compiler_params=pltpu.CompilerParams(dimension_semantics=("parallel",)),
    )(x, bn_scale.astype(jnp.bfloat16), bn_shift.astype(jnp.bfloat16),
      pack_w(w0, b0), pack_w(w1, b1))

    logits = pl.pallas_call(
        _head_body,
        out_shape=jax.ShapeDtypeStruct((B, K), jnp.float32),
    )(pooled.reshape(B, C), head_w, head_b)
    return logits


# speedup vs baseline: 1.0724x; 1.0063x over previous
import jax
import jax.numpy as jnp
from jax.experimental import pallas as pl
from jax.experimental.pallas import tpu as pltpu

_CELLS = 2
_IPS = 4


def _encoder_body(x_ref, bn_scale_ref, bn_shift_ref, w0_ref,
                  w1_ref, o_ref, *scratch):
    H = x_ref.shape[1]
    W = x_ref.shape[2]
    C = x_ref.shape[3]
    HW = H * W
    stgs = scratch[:_IPS]
    pats = scratch[_IPS:]

    lane = jax.lax.broadcasted_iota(jnp.int32, (H, W, C), 2)
    ones_lane = jnp.where(lane == 0, 1.0, 0.0).astype(jnp.bfloat16)
    for stg, pat in zip(stgs, pats):
        stg[...] = jnp.zeros(stg.shape, stg.dtype)
        pat[:, :, 9 * C:10 * C] = ones_lane

    def bn_conv(stg, pat, x2d, bn_row, w):
        scale = bn_scale_ref[bn_row:bn_row + 1, :]
        shift = bn_shift_ref[bn_row:bn_row + 1, :]
        bnx = (x2d * scale + shift).astype(jnp.bfloat16).reshape(H, W, C)
        stg[1:H + 1, :, C:2 * C] = bnx
        stg[1:H + 1, 1:W, 0:C] = bnx[:, :W - 1, :]
        stg[1:H + 1, 0:W - 1, 2 * C:3 * C] = bnx[:, 1:, :]
        pat[:, :, 0:3 * C] = stg[0:H, :, :]
        pat[:, :, 3 * C:6 * C] = stg[1:H + 1, :, :]
        pat[:, :, 6 * C:9 * C] = stg[2:H + 2, :, :]
        return jnp.dot(pat[...].reshape(HW, 10 * C), w,
                       preferred_element_type=jnp.float32)

    cells = [x_ref[i].reshape(HW, C).astype(jnp.float32) for i in range(_IPS)]
    for c in range(_CELLS):
        y0 = [bn_conv(stgs[i], pats[i], cells[i], 2 * c + 0, w0_ref[c])
              for i in range(_IPS)]
        n1 = [jnp.maximum(y0[i][:, :C], 0.0) for i in range(_IPS)]
        y1 = [bn_conv(stgs[i], pats[i], n1[i], 2 * c + 1, w1_ref[c])
              for i in range(_IPS)]
        cells = [y0[i][:, C:] + jnp.maximum(y1[i], 0.0) for i in range(_IPS)]
    for i in range(_IPS):
        o_ref[i] = jnp.sum(cells[i], axis=0, keepdims=True) * (1.0 / HW)


def _head_body(p_ref, hw_ref, hb_ref, o_ref):
    o_ref[...] = jnp.dot(p_ref[...], hw_ref[...],
                         preferred_element_type=jnp.float32) + hb_ref[...]


def kernel(x, bn_scale, bn_shift, w0, b0, w1, b1, head_w, head_b):
    x = jnp.transpose(x, (0, 2, 3, 1)).astype(jnp.bfloat16)
    B, H, W, C = x.shape
    K = head_w.shape[1]
    ten_c = 10 * C

    def pack_w(w, b):
        ext = jnp.zeros((_CELLS, C, w.shape[2]), jnp.float32)
        ext = ext.at[:, 0:1, :].set(b)
        return jnp.concatenate([w, ext], axis=1).astype(jnp.bfloat16)

    pooled = pl.pallas_call(
        _encoder_body,
        out_shape=jax.ShapeDtypeStruct((B, 1, C), jnp.float32),
        grid=(B // _IPS,),
        in_specs=[
            pl.BlockSpec((_IPS, H, W, C), lambda b: (b, 0, 0, 0)),
            pl.BlockSpec((2 * _CELLS, C), lambda b: (0, 0)),
            pl.BlockSpec((2 * _CELLS, C), lambda b: (0, 0)),
            pl.BlockSpec((_CELLS, ten_c, 2 * C), lambda b: (0, 0, 0)),
            pl.BlockSpec((_CELLS, ten_c, C), lambda b: (0, 0, 0)),
        ],
        out_specs=pl.BlockSpec((_IPS, 1, C), lambda b: (b, 0, 0)),
        scratch_shapes=(
            [pltpu.VMEM((H + 2, W, 3 * C), jnp.bfloat16) for _ in range(_IPS)]
            + [pltpu.VMEM((H, W, ten_c), jnp.bfloat16) for _ in range(_IPS)]),
        compiler_params=pltpu.CompilerParams(dimension_semantics=("parallel",)),
    )(x, bn_scale, bn_shift, pack_w(w0, b0), pack_w(w1, b1))

    logits = pl.pallas_call(
        _head_body,
        out_shape=jax.ShapeDtypeStruct((B, K), jnp.float32),
    )(pooled.reshape(B, C), head_w, head_b)
    return logits


# speedup vs baseline: 1.1132x; 1.0380x over previous
import jax
import jax.numpy as jnp
from jax.experimental import pallas as pl
from jax.experimental.pallas import tpu as pltpu

_CELLS = 2
_IPS = 8


def _encoder_body(x_ref, bn_scale_ref, bn_shift_ref, w0_ref, b0_ref,
                  w1_ref, b1_ref, o_ref, *scratch):
    H = x_ref.shape[1]
    W = x_ref.shape[2]
    C = x_ref.shape[3]
    HW = H * W
    stgs = scratch[:_IPS]
    pats = scratch[_IPS:]

    for stg in stgs:
        stg[...] = jnp.zeros(stg.shape, stg.dtype)

    def bn_conv(stg, pat, x2d, bn_row, w, b):
        scale = bn_scale_ref[bn_row:bn_row + 1, :]
        shift = bn_shift_ref[bn_row:bn_row + 1, :]
        bnx = (x2d * scale + shift).astype(jnp.bfloat16).reshape(H, W, C)
        stg[1:H + 1, :, C:2 * C] = bnx
        stg[1:H + 1, 1:W, 0:C] = bnx[:, :W - 1, :]
        stg[1:H + 1, 0:W - 1, 2 * C:3 * C] = bnx[:, 1:, :]
        pat[:, :, 0:3 * C] = stg[0:H, :, :]
        pat[:, :, 3 * C:6 * C] = stg[1:H + 1, :, :]
        pat[:, :, 6 * C:9 * C] = stg[2:H + 2, :, :]
        return jnp.dot(pat[...].reshape(HW, 9 * C), w,
                       preferred_element_type=jnp.float32) + b

    cells = [x_ref[i].reshape(HW, C).astype(jnp.float32) for i in range(_IPS)]
    for c in range(_CELLS):
        y0 = [bn_conv(stgs[i], pats[i], cells[i], 2 * c + 0,
                      w0_ref[c], b0_ref[c]) for i in range(_IPS)]
        n1 = [jnp.maximum(y0[i][:, :C], 0.0) for i in range(_IPS)]
        y1 = [bn_conv(stgs[i], pats[i], n1[i], 2 * c + 1,
                      w1_ref[c], b1_ref[c]) for i in range(_IPS)]
        cells = [y0[i][:, C:] + jnp.maximum(y1[i], 0.0) for i in range(_IPS)]
    for i in range(_IPS):
        o_ref[i] = jnp.sum(cells[i], axis=0, keepdims=True) * (1.0 / HW)


def _head_body(p_ref, hw_ref, hb_ref, o_ref):
    o_ref[...] = jnp.dot(p_ref[...], hw_ref[...],
                         preferred_element_type=jnp.float32) + hb_ref[...]


def kernel(x, bn_scale, bn_shift, w0, b0, w1, b1, head_w, head_b):
    x = jnp.transpose(x, (0, 2, 3, 1)).astype(jnp.bfloat16)
    B, H, W, C = x.shape
    K = head_w.shape[1]
    nine_c = 9 * C

    pooled = pl.pallas_call(
        _encoder_body,
        out_shape=jax.ShapeDtypeStruct((B, 1, C), jnp.float32),
        grid=(B // _IPS,),
        in_specs=[
            pl.BlockSpec((_IPS, H, W, C), lambda b: (b, 0, 0, 0)),
            pl.BlockSpec((2 * _CELLS, C), lambda b: (0, 0)),
            pl.BlockSpec((2 * _CELLS, C), lambda b: (0, 0)),
            pl.BlockSpec((_CELLS, nine_c, 2 * C), lambda b: (0, 0, 0)),
            pl.BlockSpec((_CELLS, 1, 2 * C), lambda b: (0, 0, 0)),
            pl.BlockSpec((_CELLS, nine_c, C), lambda b: (0, 0, 0)),
            pl.BlockSpec((_CELLS, 1, C), lambda b: (0, 0, 0)),
        ],
        out_specs=pl.BlockSpec((_IPS, 1, C), lambda b: (b, 0, 0)),
        scratch_shapes=(
            [pltpu.VMEM((H + 2, W, 3 * C), jnp.bfloat16) for _ in range(_IPS)]
            + [pltpu.VMEM((H, W, nine_c), jnp.bfloat16) for _ in range(_IPS)]),
        compiler_params=pltpu.CompilerParams(dimension_semantics=("parallel",)),
    )(x, bn_scale, bn_shift, w0.astype(jnp.bfloat16), b0,
      w1.astype(jnp.bfloat16), b1)

    logits = pl.pallas_call(
        _head_body,
        out_shape=jax.ShapeDtypeStruct((B, K), jnp.float32),
    )(pooled.reshape(B, C), head_w, head_b)
    return logits


# speedup vs baseline: 1.1588x; 1.0409x over previous
import jax
import jax.numpy as jnp
from jax.experimental import pallas as pl
from jax.experimental.pallas import tpu as pltpu

_CELLS = 2
_IPS = 8


def _encoder_body(x_ref, bn_scale_ref, bn_shift_ref, w0_ref, b0_ref,
                  w1_ref, b1_ref, o_ref, *scratch):
    H = x_ref.shape[1]
    W = x_ref.shape[2]
    C = x_ref.shape[3]
    HW = H * W
    stgs = scratch
    rpc = H // 4

    for stg in stgs:
        stg[...] = jnp.zeros(stg.shape, stg.dtype)

    def bn_conv(stg, x2d, bn_row, w_ref, c, b):
        scale = bn_scale_ref[bn_row:bn_row + 1, :]
        shift = bn_shift_ref[bn_row:bn_row + 1, :]
        bnx = (x2d * scale + shift).astype(jnp.bfloat16).reshape(H, W, C)
        stg[1:H + 1, :, C:2 * C] = bnx
        stg[1:H + 1, 1:W, 0:C] = bnx[:, :W - 1, :]
        stg[1:H + 1, 0:W - 1, 2 * C:3 * C] = bnx[:, 1:, :]
        outs = []
        for j in range(4):
            acc = b
            for kh in range(3):
                lhs = stg[kh + rpc * j:kh + rpc * j + rpc, :, :]
                wk = w_ref[c, kh * 3 * C:(kh + 1) * 3 * C, :]
                acc = acc + jnp.dot(lhs.reshape(rpc * W, 3 * C), wk,
                                    preferred_element_type=jnp.float32)
            outs.append(acc)
        return jnp.concatenate(outs, axis=0)

    cells = [x_ref[i].reshape(HW, C).astype(jnp.float32) for i in range(_IPS)]
    for c in range(_CELLS):
        y0 = [bn_conv(stgs[i], cells[i], 2 * c + 0,
                      w0_ref, c, b0_ref[c]) for i in range(_IPS)]
        n1 = [jnp.maximum(y0[i][:, :C], 0.0) for i in range(_IPS)]
        y1 = [bn_conv(stgs[i], n1[i], 2 * c + 1,
                      w1_ref, c, b1_ref[c]) for i in range(_IPS)]
        cells = [y0[i][:, C:] + jnp.maximum(y1[i], 0.0) for i in range(_IPS)]
    for i in range(_IPS):
        o_ref[i] = jnp.sum(cells[i], axis=0, keepdims=True) * (1.0 / HW)


def _head_body(p_ref, hw_ref, hb_ref, o_ref):
    o_ref[...] = jnp.dot(p_ref[...], hw_ref[...],
                         preferred_element_type=jnp.float32) + hb_ref[...]


def kernel(x, bn_scale, bn_shift, w0, b0, w1, b1, head_w, head_b):
    x = jnp.transpose(x, (0, 2, 3, 1)).astype(jnp.bfloat16)
    B, H, W, C = x.shape
    K = head_w.shape[1]
    nine_c = 9 * C

    pooled = pl.pallas_call(
        _encoder_body,
        out_shape=jax.ShapeDtypeStruct((B, 1, C), jnp.float32),
        grid=(B // _IPS,),
        in_specs=[
            pl.BlockSpec((_IPS, H, W, C), lambda b: (b, 0, 0, 0)),
            pl.BlockSpec((2 * _CELLS, C), lambda b: (0, 0)),
            pl.BlockSpec((2 * _CELLS, C), lambda b: (0, 0)),
            pl.BlockSpec((_CELLS, nine_c, 2 * C), lambda b: (0, 0, 0)),
            pl.BlockSpec((_CELLS, 1, 2 * C), lambda b: (0, 0, 0)),
            pl.BlockSpec((_CELLS, nine_c, C), lambda b: (0, 0, 0)),
            pl.BlockSpec((_CELLS, 1, C), lambda b: (0, 0, 0)),
        ],
        out_specs=pl.BlockSpec((_IPS, 1, C), lambda b: (b, 0, 0)),
        scratch_shapes=[pltpu.VMEM((H + 2, W, 3 * C), jnp.bfloat16)
                        for _ in range(_IPS)],
        compiler_params=pltpu.CompilerParams(dimension_semantics=("parallel",)),
    )(x, bn_scale, bn_shift, w0.astype(jnp.bfloat16), b0,
      w1.astype(jnp.bfloat16), b1)

    logits = pl.pallas_call(
        _head_body,
        out_shape=jax.ShapeDtypeStruct((B, K), jnp.float32),
    )(pooled.reshape(B, C), head_w, head_b)
    return logits
